# baseline (speedup 1.0000x reference)
import jax
import jax.numpy as jnp
from jax import lax
from jax.experimental import pallas as pl
from jax.experimental.pallas import tpu as pltpu
from jax.experimental.pallas import tpu_sc as plsc

NUM_CORES = 2
NUM_SUBCORES = 16
LANES = 16
NW = NUM_CORES * NUM_SUBCORES

ROWS_PER_CHUNK = 4
COL_SPAN = 6256


def _make_sc_kernel(B, D, E, S):
    assert B % NW == 0 and D % ROWS_PER_CHUNK == 0
    bpw = B // NW
    n_chunks = D // ROWS_PER_CHUNK
    n_groups = bpw // LANES
    tail_span = E - (NUM_SUBCORES - 1) * COL_SPAN
    assert 0 < tail_span <= COL_SPAN
    mesh = plsc.VectorSubcoreMesh(core_axis_name="c", subcore_axis_name="s")

    def body(pairs_hbm, ewt_hbm, swt_hbm, out_hbm,
             eids_v, sids_v, swt_v, ev_v, acc_v, out_v,
             spbufs, sem_sp0, sem_sp1, sem_ev, sem_sw):
        sid = lax.axis_index("s")
        wid = sid * NUM_CORES + lax.axis_index("c")
        base = wid * bpw
        col0 = sid * COL_SPAN
        sems = [sem_sp0, sem_sp1]

        sw_cp = pltpu.async_copy(swt_hbm, swt_v, sem_sw)

        pltpu.sync_copy(pairs_hbm.at[0, pl.ds(base, bpw)], eids_v)
        pltpu.sync_copy(pairs_hbm.at[1, pl.ds(base, bpw)], sids_v)
        for c in range(n_groups):
            sl = pl.ds(c * LANES, LANES)
            eids_v[sl] = jnp.minimum(jnp.maximum(eids_v[sl], 0), E - 1)
            sids_v[sl] = jnp.minimum(jnp.maximum(sids_v[sl], 0), S - 1)

        def stripe_args(c, span):
            rows = pl.ds(c * ROWS_PER_CHUNK, ROWS_PER_CHUNK)
            return (ewt_hbm.at[rows, pl.ds(col0, span)],
                    spbufs.at[c % 2].at[:, pl.ds(col0, span)],
                    sems[c % 2])

        def stage(c):
            @pl.when(sid < NUM_SUBCORES - 1)
            def _():
                pltpu.async_copy(*stripe_args(c, COL_SPAN))

            @pl.when(sid == NUM_SUBCORES - 1)
            def _():
                pltpu.async_copy(*stripe_args(c, tail_span))

        def wait_stage(c):
            @pl.when(sid < NUM_SUBCORES - 1)
            def _():
                pltpu.make_async_copy(*stripe_args(c, COL_SPAN)).wait()

            @pl.when(sid == NUM_SUBCORES - 1)
            def _():
                pltpu.make_async_copy(*stripe_args(c, tail_span)).wait()

        stage(0)
        sw_cp.wait()

        for c in range(n_chunks):
            wait_stage(c)
            plsc.subcore_barrier()
            if c + 1 < n_chunks:
                stage(c + 1)

            ev_cps = [
                pltpu.async_copy(spbufs.at[c % 2].at[j].at[eids_v],
                                 ev_v.at[j], sem_ev)
                for j in range(ROWS_PER_CHUNK)
            ]
            for cp in ev_cps:
                cp.wait()

            def g_body(g, carry, c=c):
                i0 = g * LANES
                sid_vec = sids_v[pl.ds(i0, LANES)]
                acc = acc_v[pl.ds(i0, LANES)] if c else jnp.zeros((LANES,), jnp.float32)
                for j in range(ROWS_PER_CHUNK):
                    d = c * ROWS_PER_CHUNK + j
                    ev = ev_v[j, pl.ds(i0, LANES)]
                    sv = plsc.load_gather(
                        swt_v, [jnp.full((LANES,), d, jnp.int32), sid_vec])
                    acc = acc + ev * sv
                if c + 1 < n_chunks:
                    acc_v[pl.ds(i0, LANES)] = acc
                else:
                    out_v[pl.ds(i0, LANES)] = 1.0 / (1.0 + jnp.exp(-acc))
                return carry

            lax.fori_loop(0, n_groups, g_body, 0)

        pltpu.sync_copy(out_v, out_hbm.at[pl.ds(base, bpw)])

    return pl.kernel(
        body,
        out_type=jax.ShapeDtypeStruct((B,), jnp.float32),
        mesh=mesh,
        compiler_params=pltpu.CompilerParams(
            needs_layout_passes=False, use_tc_tiling_on_sc=False),
        scratch_types=[
            pltpu.VMEM((bpw,), jnp.int32),
            pltpu.VMEM((bpw,), jnp.int32),
            pltpu.VMEM((D, S), jnp.float32),
            pltpu.VMEM((ROWS_PER_CHUNK, bpw), jnp.float32),
            pltpu.VMEM((bpw,), jnp.float32),
            pltpu.VMEM((bpw,), jnp.float32),
            pltpu.VMEM_SHARED((2, ROWS_PER_CHUNK, E), jnp.float32),
            pltpu.SemaphoreType.DMA,
            pltpu.SemaphoreType.DMA,
            pltpu.SemaphoreType.DMA,
            pltpu.SemaphoreType.DMA,
        ],
    )


def kernel(pairs, exercise_w, skill_w):
    B = pairs.shape[0]
    E, D = exercise_w.shape
    S = skill_w.shape[0]
    sc = _make_sc_kernel(B, D, E, S)
    return sc(pairs.T, exercise_w.T, skill_w.T)
